# baseline (speedup 1.0000x reference)
import functools

import jax
import jax.numpy as jnp
from jax import lax
from jax.experimental import pallas as pl
from jax.experimental.pallas import tpu as pltpu
from jax.experimental.pallas import tpu_sc as plsc


def _mm_body(p_hbm, w_ref, b_ref, o_ref, pv, sem):
    i = pl.program_id(0)
    br = o_ref.shape[0]
    cp = pltpu.make_async_copy(p_hbm.at[:, pl.ds(i * br, br), :], pv, sem)
    cp.start()
    cp.wait()
    agg = (pv[0].astype(jnp.float32)
           + pv[1].astype(jnp.float32)).astype(jnp.bfloat16)
    o_ref[...] = (jnp.dot(agg, w_ref[...].astype(jnp.bfloat16),
                          preferred_element_type=jnp.float32)
                  + b_ref[...])


def _matmul_bias(partials, w, bias2d, n):
    _, np_, d = partials.shape
    _, m = w.shape
    br = 400
    assert n % br == 0 and np_ >= n
    return pl.pallas_call(
        _mm_body,
        grid=(n // br,),
        in_specs=[
            pl.BlockSpec(memory_space=pltpu.HBM),
            pl.BlockSpec((d, m), lambda i: (0, 0)),
            pl.BlockSpec((1, m), lambda i: (0, 0)),
        ],
        out_specs=pl.BlockSpec((br, m), lambda i: (i, 0)),
        out_shape=jax.ShapeDtypeStruct((n, m), jnp.float32),
        scratch_shapes=[pltpu.VMEM((2, br, d), jnp.bfloat16),
                        pltpu.SemaphoreType.DMA],
    )(partials, w, bias2d)


def _sc_aggregate(sup, sd_hbm_arr, w_hbm_arr, nk, ch):
    n, h = sup.shape
    nw, ec = sd_hbm_arr.shape
    info = plsc.get_sparse_core_info()
    nc, ns = info.num_cores, info.num_subcores
    assert nw == nc * ns and h % 32 == 0 and ch % 8 == 0 and n < 2 ** 16
    assert ec == nk * ch and w_hbm_arr.shape == (nw, ec)
    assert 14 * 624 + 2 * 632 == n

    mesh = plsc.VectorSubcoreMesh(core_axis_name="c", subcore_axis_name="s")

    @functools.partial(
        pl.kernel,
        mesh=mesh,
        compiler_params=pltpu.CompilerParams(use_tc_tiling_on_sc=False,
                                             needs_layout_passes=False),
        out_type=jax.ShapeDtypeStruct((nc, n, h), jnp.bfloat16),
        scratch_types=[
            pltpu.VMEM((nk * ch,), jnp.int32),
            pltpu.VMEM((nk, ch), jnp.int32),
            pltpu.VMEM((nk, ch), jnp.int32),
            pltpu.VMEM((nk * ch,), jnp.int32),
            [pltpu.VMEM((ch, h), jnp.bfloat16)] * 8,
            pltpu.VMEM_SHARED((n, h), jnp.bfloat16),
            [pltpu.SemaphoreType.DMA] * 8,
            [pltpu.SemaphoreType.DMA] * 8,
        ],
    )
    def agg(sup_hbm, sd_hbm, w_hbm, out_hbm,
            sd, sidx, didx, wv, rows, acc, gsem, ssem):
        c = lax.axis_index("c")
        s = lax.axis_index("s")
        tid = c * ns + s

        pltpu.sync_copy(sd_hbm.at[tid], sd)
        pltpu.sync_copy(w_hbm.at[tid], wv)

        zb = jnp.zeros((32,), jnp.bfloat16)

        @plsc.parallel_loop(0, ch)
        def zero_rows(r):
            for j in range(h // 32):
                rows[0][r, pl.ds(j * 32, 32)] = zb

        def zero_acc(r0, rpt):
            for i in range(rpt // ch):
                pltpu.sync_copy(rows[0], acc.at[pl.ds(r0 + i * ch, ch)])
            t = rpt % ch
            if t:
                pltpu.sync_copy(rows[0].at[pl.ds(0, t)],
                                acc.at[pl.ds(r0 + (rpt // ch) * ch, t)])

        @pl.when(s < 14)
        def _():
            zero_acc(s * 624, 624)

        @pl.when(s >= 14)
        def _():
            zero_acc(14 * 624 + (s - 14) * 632, 632)

        @plsc.parallel_loop(0, nk)
        def unpack_idx(k):
            for g in range(ch // 16):
                sl = pl.ds(g * 16, 16)
                v = sd[pl.ds(k * ch + g * 16, 16)]
                sidx[k, sl] = v & 0xFFFF
                didx[k, sl] = lax.shift_right_logical(v, 16)

        plsc.subcore_barrier()

        nbuf = len(rows)

        def start_gather(k, b):
            pltpu.async_copy(sup_hbm.at[sidx.at[k]], rows[b], gsem[b])

        def wait_gather(k, b):
            pltpu.make_async_copy(sup_hbm.at[sidx.at[k]], rows[b],
                                  gsem[b]).wait()

        def scale(k, b):
            @plsc.parallel_loop(0, ch // 16, unroll=2)
            def scale_body(g):
                wg = wv[pl.ds(k * ch + g * 16, 16)]
                for l in range(16):
                    wsplat = plsc.bitcast(jnp.broadcast_to(wg[l], (16,)),
                                          jnp.bfloat16)
                    e = g * 16 + l
                    for j in range(h // 32):
                        sl = pl.ds(j * 32, 32)
                        rows[b][e, sl] = rows[b][e, sl] * wsplat

        def start_scatter(k, b):
            pltpu.async_copy(rows[b], acc.at[didx.at[k]], ssem[b], add=True)

        def wait_scatter(k, b):
            pltpu.make_async_copy(rows[b], acc.at[didx.at[k]],
                                  ssem[b]).wait()

        for b in range(nbuf):
            start_gather(b, b)

        nq, rem = divmod(nk, nbuf)

        def ring_body(q, carry):
            kx = q * nbuf
            for b in range(nbuf):
                wait_gather(kx + b, b)
                scale(kx + b, b)
                start_scatter(kx + b, b)
            for b in range(nbuf):
                wait_scatter(kx + b, b)

                @pl.when(kx + b + nbuf < nk)
                def _(b=b):
                    start_gather(kx + b + nbuf, b)

            return carry

        lax.fori_loop(0, nq, ring_body, 0)
        for b in range(rem):
            kx = nq * nbuf + b
            wait_gather(kx, b)
            scale(kx, b)
            start_scatter(kx, b)
        for b in range(rem):
            wait_scatter(nq * nbuf + b, b)
        plsc.subcore_barrier()

        @pl.when(s < 14)
        def _():
            r0 = s * 624
            pltpu.sync_copy(acc.at[pl.ds(r0, 624)],
                            out_hbm.at[c, pl.ds(r0, 624)])

        @pl.when(s >= 14)
        def _():
            r0 = 14 * 624 + (s - 14) * 632
            pltpu.sync_copy(acc.at[pl.ds(r0, 632)],
                            out_hbm.at[c, pl.ds(r0, 632)])

    return agg(sup, sd_hbm_arr, w_hbm_arr)


def kernel(inputs, edge_index, edge_weight, weight, bias):
    n, d_in = inputs.shape
    e = edge_index.shape[1]
    d_out = weight.shape[1]

    nw = 32
    ch = 80
    assert e % (nw * ch) == 0
    nk = e // (nw * ch)

    xbf = inputs.astype(jnp.bfloat16)

    sd = (edge_index[0] | (edge_index[1] << 16)).reshape(nw, nk * ch)
    wb = lax.bitcast_convert_type(edge_weight.astype(jnp.bfloat16),
                                  jnp.uint16).astype(jnp.int32)
    wbits = (wb | (wb << 16)).reshape(nw, nk * ch)

    partials = _sc_aggregate(xbf, sd, wbits, nk, ch)

    return _matmul_bias(partials, weight, bias.reshape(1, d_out), n)


# speedup vs baseline: 1.1340x; 1.1340x over previous
import functools

import jax
import jax.numpy as jnp
from jax import lax
from jax.experimental import pallas as pl
from jax.experimental.pallas import tpu as pltpu
from jax.experimental.pallas import tpu_sc as plsc


def _mm_body(p_ref, w_ref, b_ref, o_ref):
    agg = (p_ref[0].astype(jnp.float32)
           + p_ref[1].astype(jnp.float32)).astype(jnp.bfloat16)
    o_ref[...] = (jnp.dot(agg, w_ref[...].astype(jnp.bfloat16),
                          preferred_element_type=jnp.float32)
                  + b_ref[...])


def _matmul_bias(partials, w, bias2d, n):
    _, np_, d = partials.shape
    _, m = w.shape
    br = 400
    assert n % br == 0 and np_ >= n
    return pl.pallas_call(
        _mm_body,
        grid=(n // br,),
        in_specs=[
            pl.BlockSpec((2, br, d), lambda i: (0, i, 0)),
            pl.BlockSpec((d, m), lambda i: (0, 0)),
            pl.BlockSpec((1, m), lambda i: (0, 0)),
        ],
        out_specs=pl.BlockSpec((br, m), lambda i: (i, 0)),
        out_shape=jax.ShapeDtypeStruct((n, m), jnp.float32),
    )(partials, w, bias2d)


def _sc_aggregate(sup, ei, wi, nk, ch):
    n, h = sup.shape
    _, e = ei.shape
    info = plsc.get_sparse_core_info()
    nc, ns = info.num_cores, info.num_subcores
    nw = nc * ns
    ec = nk * ch
    assert h % 32 == 0 and ch % 8 == 0 and e == nw * ec
    assert wi.shape == (e,) and wi.dtype == jnp.int32
    assert 14 * 624 + 2 * 632 == n

    mesh = plsc.VectorSubcoreMesh(core_axis_name="c", subcore_axis_name="s")

    @functools.partial(
        pl.kernel,
        mesh=mesh,
        compiler_params=pltpu.CompilerParams(use_tc_tiling_on_sc=False,
                                             needs_layout_passes=False),
        out_type=jax.ShapeDtypeStruct((nc, n, h), jnp.bfloat16),
        scratch_types=[
            pltpu.VMEM((nk * ch,), jnp.int32),
            pltpu.VMEM((nk * ch,), jnp.int32),
            pltpu.VMEM((nk, ch), jnp.int32),
            pltpu.VMEM((nk * ch,), jnp.int32),
            [pltpu.VMEM((ch, h), jnp.bfloat16)] * 8,
            pltpu.VMEM_SHARED((n, h), jnp.bfloat16),
            [pltpu.SemaphoreType.DMA] * 8,
            [pltpu.SemaphoreType.DMA] * 8,
        ],
    )
    def agg(sup_hbm, ei_hbm, wi_hbm, out_hbm,
            sidx, dbuf, didx, wv, rows, acc, gsem, ssem):
        c = lax.axis_index("c")
        s = lax.axis_index("s")
        tid = c * ns + s

        e0 = tid * (nk * ch)
        pltpu.sync_copy(ei_hbm.at[0, pl.ds(e0, nk * ch)], sidx)
        pltpu.sync_copy(ei_hbm.at[1, pl.ds(e0, nk * ch)], dbuf)
        pltpu.sync_copy(wi_hbm.at[pl.ds(e0, nk * ch)], wv)

        zb = jnp.zeros((32,), jnp.bfloat16)

        @plsc.parallel_loop(0, ch)
        def zero_rows(r):
            for j in range(h // 32):
                rows[0][r, pl.ds(j * 32, 32)] = zb

        def zero_acc(r0, rpt):
            for i in range(rpt // ch):
                pltpu.sync_copy(rows[0], acc.at[pl.ds(r0 + i * ch, ch)])
            t = rpt % ch
            if t:
                pltpu.sync_copy(rows[0].at[pl.ds(0, t)],
                                acc.at[pl.ds(r0 + (rpt // ch) * ch, t)])

        @pl.when(s < 14)
        def _():
            zero_acc(s * 624, 624)

        @pl.when(s >= 14)
        def _():
            zero_acc(14 * 624 + (s - 14) * 632, 632)

        @plsc.parallel_loop(0, nk)
        def fill_didx(k):
            for g in range(ch // 16):
                didx[k, pl.ds(g * 16, 16)] = dbuf[pl.ds(k * ch + g * 16, 16)]

        plsc.subcore_barrier()

        nbuf = len(rows)

        def start_gather(k, b):
            pltpu.async_copy(sup_hbm.at[sidx.at[pl.ds(k * ch, ch)]],
                             rows[b], gsem[b])

        def wait_gather(k, b):
            pltpu.make_async_copy(sup_hbm.at[sidx.at[pl.ds(k * ch, ch)]],
                                  rows[b], gsem[b]).wait()

        def scale(k, b):
            @plsc.parallel_loop(0, ch // 16, unroll=2)
            def scale_body(g):
                wf = wv[pl.ds(k * ch + g * 16, 16)]
                t = (wf + 0x8000) & jnp.int32(-65536)
                wg = t | lax.shift_right_logical(t, 16)
                for l in range(16):
                    wsplat = plsc.bitcast(jnp.broadcast_to(wg[l], (16,)),
                                          jnp.bfloat16)
                    e = g * 16 + l
                    for j in range(h // 32):
                        sl = pl.ds(j * 32, 32)
                        rows[b][e, sl] = rows[b][e, sl] * wsplat

        def start_scatter(k, b):
            pltpu.async_copy(rows[b], acc.at[didx.at[k]], ssem[b], add=True)

        def wait_scatter(k, b):
            pltpu.make_async_copy(rows[b], acc.at[didx.at[k]],
                                  ssem[b]).wait()

        for b in range(nbuf):
            start_gather(b, b)

        nq, rem = divmod(nk, nbuf)

        def ring_body(q, carry):
            kx = q * nbuf
            for b in range(nbuf):
                wait_gather(kx + b, b)
                scale(kx + b, b)
                start_scatter(kx + b, b)
            for b in range(nbuf):
                wait_scatter(kx + b, b)

                @pl.when(kx + b + nbuf < nk)
                def _(b=b):
                    start_gather(kx + b + nbuf, b)

            return carry

        lax.fori_loop(0, nq, ring_body, 0)
        for b in range(rem):
            kx = nq * nbuf + b
            wait_gather(kx, b)
            scale(kx, b)
            start_scatter(kx, b)
        for b in range(rem):
            wait_scatter(nq * nbuf + b, b)
        plsc.subcore_barrier()

        @pl.when(s < 14)
        def _():
            r0 = s * 624
            pltpu.sync_copy(acc.at[pl.ds(r0, 624)],
                            out_hbm.at[c, pl.ds(r0, 624)])

        @pl.when(s >= 14)
        def _():
            r0 = 14 * 624 + (s - 14) * 632
            pltpu.sync_copy(acc.at[pl.ds(r0, 632)],
                            out_hbm.at[c, pl.ds(r0, 632)])

    return agg(sup, ei, wi)


def kernel(inputs, edge_index, edge_weight, weight, bias):
    n, d_in = inputs.shape
    e = edge_index.shape[1]
    d_out = weight.shape[1]

    nw = 32
    ch = 80
    assert e % (nw * ch) == 0
    nk = e // (nw * ch)

    xbf = inputs.astype(jnp.bfloat16)

    wi = lax.bitcast_convert_type(edge_weight, jnp.int32)

    partials = _sc_aggregate(xbf, edge_index, wi, nk, ch)

    return _matmul_bias(partials, weight, bias.reshape(1, d_out), n)
